# baseline (speedup 1.0000x reference)
import functools

import jax
import jax.numpy as jnp
from jax import lax
from jax.experimental import pallas as pl
from jax.experimental.pallas import tpu as pltpu
from jax.experimental.pallas import tpu_sc as plsc

_N_NODES = 10000
_N_EDGES = 320000
_NODE_DIM = 128
_E_DIM = 16
_E_VOCAB = 16

_INFO = plsc.get_sparse_core_info()
_NC, _NS = _INFO.num_cores, _INFO.num_subcores
_NW = _NC * _NS

_NODE_CHUNK = 320

_EB = 6400
_N_EB = _N_EDGES // _EB


def _sc_node_lookup(node_ids, node_table):
    mesh = plsc.VectorSubcoreMesh(core_axis_name="c", subcore_axis_name="s")

    @functools.partial(
        pl.kernel,
        mesh=mesh,
        out_type=jax.ShapeDtypeStruct((_N_NODES, _NODE_DIM), jnp.float32),
        scratch_types=[
            pltpu.VMEM((_NODE_CHUNK,), jnp.int32),
            pltpu.VMEM((_NODE_CHUNK, _NODE_DIM), jnp.float32),
            pltpu.SemaphoreType.DMA,
        ],
        compiler_params=pltpu.CompilerParams(needs_layout_passes=False),
    )
    def k(node_ids_hbm, node_tab_hbm, node_out, nidx_v, nrows_v, sem):
        wid = lax.axis_index("s") * _NC + lax.axis_index("c")
        nbase = jnp.where(wid < _NW - 1,
                          wid * _NODE_CHUNK, _N_NODES - _NODE_CHUNK)
        pltpu.sync_copy(node_ids_hbm.at[pl.ds(nbase, _NODE_CHUNK)], nidx_v)
        pltpu.async_copy(node_tab_hbm.at[nidx_v], nrows_v, sem).wait()
        pltpu.sync_copy(nrows_v, node_out.at[pl.ds(nbase, _NODE_CHUNK)])

    return k(node_ids, node_table)


def _tc_edge_body(eids_hbm, etab_ref, out_ref, ids_v, sems):
    b = pl.program_id(0)

    def start(slot, blk):
        return pltpu.make_async_copy(
            eids_hbm.at[pl.ds(blk * _EB, _EB)], ids_v.at[slot], sems.at[slot])

    @pl.when(b == 0)
    def _():
        start(0, 0).start()

    @pl.when(b + 1 < _N_EB)
    def _():
        start((b + 1) % 2, b + 1).start()

    start(b % 2, b).wait()
    ids = ids_v[b % 2]
    onehot = (lax.broadcasted_iota(jnp.int32, (_E_VOCAB, _EB), 0)
              == ids[None, :]).astype(jnp.float32)
    out_ref[...] = lax.dot_general(
        etab_ref[...], onehot, (((0,), (0,)), ((), ())),
        preferred_element_type=jnp.float32)


def _tc_edge_lookup(edge_ids, edge_table):
    return pl.pallas_call(
        _tc_edge_body,
        grid=(_N_EB,),
        in_specs=[
            pl.BlockSpec(memory_space=pl.ANY),
            pl.BlockSpec((_E_VOCAB, _E_DIM), lambda b: (0, 0)),
        ],
        out_specs=pl.BlockSpec((_E_VOCAB, _EB), lambda b: (0, b)),
        out_shape=jax.ShapeDtypeStruct((_E_DIM, _N_EDGES), jnp.float32),
        scratch_shapes=[
            pltpu.VMEM((2, _EB), jnp.int32),
            pltpu.SemaphoreType.DMA((2,)),
        ],
        compiler_params=pltpu.CompilerParams(
            dimension_semantics=("arbitrary",)),
    )(edge_ids, edge_table)


def kernel(node_ids, edge_ids, node_table, edge_table):
    node_feat = _sc_node_lookup(node_ids, node_table)
    edge_t = _tc_edge_lookup(edge_ids, edge_table)
    return node_feat, edge_t.T


# speedup vs baseline: 1.1982x; 1.1982x over previous
import functools

import jax
import jax.numpy as jnp
from jax import lax
from jax.experimental import pallas as pl
from jax.experimental.pallas import tpu as pltpu
from jax.experimental.pallas import tpu_sc as plsc

_N_NODES = 10000
_N_EDGES = 320000
_NODE_DIM = 128
_E_DIM = 16
_E_VOCAB = 16

_INFO = plsc.get_sparse_core_info()
_NC, _NS = _INFO.num_cores, _INFO.num_subcores
_NW = _NC * _NS
_L = 16

_NODE_CHUNK = 320

_BLK = 128
_N_BLKS = _N_EDGES // _BLK
_NB_W = 79
_TB = 16
_CHUNK_SIZES = (16, 16, 16, 16, 15)
_CBUF_PLANE = _TB * _BLK * 8
_PLANE_STRIDE = _N_BLKS * _BLK * 8
_UNROLL = 8


def _sc_lookup(node_ids, edge_ids, node_table, edge_table):
    mesh = plsc.VectorSubcoreMesh(core_axis_name="c", subcore_axis_name="s")

    @functools.partial(
        pl.kernel,
        mesh=mesh,
        out_type=(
            jax.ShapeDtypeStruct((_N_NODES, _NODE_DIM), jnp.float32),
            jax.ShapeDtypeStruct((_N_EDGES * _E_DIM,), jnp.float32),
        ),
        scratch_types=[
            pltpu.VMEM((_NODE_CHUNK,), jnp.int32),
            pltpu.VMEM((_NODE_CHUNK, _NODE_DIM), jnp.float32),
            pltpu.VMEM((_TB * _BLK,), jnp.int32),
            pltpu.VMEM((_TB * _BLK,), jnp.int32),
            pltpu.VMEM((2 * _CBUF_PLANE,), jnp.float32),
            pltpu.VMEM((2 * _CBUF_PLANE,), jnp.float32),
            pltpu.VMEM((_E_VOCAB * _E_DIM * _L,), jnp.float32),
            pltpu.SemaphoreType.DMA,
            pltpu.SemaphoreType.DMA,
            pltpu.SemaphoreType.DMA,
            pltpu.SemaphoreType.DMA,
            pltpu.SemaphoreType.DMA,
            pltpu.SemaphoreType.DMA,
            pltpu.SemaphoreType.DMA,
        ],
        compiler_params=pltpu.CompilerParams(needs_layout_passes=False),
    )
    def k(node_ids_hbm, edge_ids_hbm, node_tab_hbm, edge_tab_hbm,
          node_out, edge_out, nidx_v, nrows_v, eidx0, eidx1, cbuf0, cbuf1,
          etab_v, sn_id, sn_g, sn_w, se_id0, se_id1, se_w0, se_w1):
        wid = lax.axis_index("s") * _NC + lax.axis_index("c")
        nbase = jnp.where(wid < _NW - 1,
                          wid * _NODE_CHUNK, _N_NODES - _NODE_CHUNK)
        bstart = (wid * _N_BLKS) // _NW

        nid_cp = pltpu.async_copy(
            node_ids_hbm.at[pl.ds(nbase, _NODE_CHUNK)], nidx_v, sn_id)
        eidx = (eidx0, eidx1)
        cbuf = (cbuf0, cbuf1)
        seid = (se_id0, se_id1)
        sew = (se_w0, se_w1)
        csum = [sum(_CHUNK_SIZES[:n]) for n in range(len(_CHUNK_SIZES))]
        ecp = [
            pltpu.async_copy(
                edge_ids_hbm.at[pl.ds((bstart + csum[c]) * _BLK,
                                      _CHUNK_SIZES[c] * _BLK)],
                eidx[c].at[pl.ds(0, _CHUNK_SIZES[c] * _BLK)], seid[c])
            for c in range(2)
        ]
        pltpu.sync_copy(edge_tab_hbm, etab_v)
        nid_cp.wait()
        ng = pltpu.async_copy(node_tab_hbm.at[nidx_v], nrows_v, sn_g)

        lanes = lax.iota(jnp.int32, _L)

        def expand_chunk(ids_ref, rows_ref, n_groups):
            def gather_group(g):
                idvec = ids_ref[pl.ds(g * _L, _L)]
                src = idvec * (_E_DIM * _L) + lanes
                return tuple(plsc.load_gather(etab_v, [src + j * _L])
                             for j in range(_E_DIM))

            def store_group(g, vals):
                dstb = (g // 8) * (8 * _BLK) + (g % 8) * _L
                for j in range(_E_DIM):
                    off = dstb + (j // 8) * _CBUF_PLANE + (j % 8) * _BLK
                    rows_ref[pl.ds(off, _L)] = vals[j]

            def body(it, _):
                g0 = it * _UNROLL
                prev = gather_group(g0)
                for u in range(1, _UNROLL):
                    cur = gather_group(g0 + u)
                    store_group(g0 + u - 1, prev)
                    prev = cur
                store_group(g0 + _UNROLL - 1, prev)
                return 0

            lax.fori_loop(0, n_groups // _UNROLL, body, 0)

        wr = [(), ()]
        nw = None
        n_chunks = len(_CHUNK_SIZES)
        for c in range(n_chunks):
            b = c & 1
            tb = _CHUNK_SIZES[c]
            ecp[b].wait()
            for h in wr[b]:
                h.wait()
            expand_chunk(eidx[b], cbuf[b], tb * 8)
            if c + 2 < n_chunks:
                nxt = c + 2
                ecp[b] = pltpu.async_copy(
                    edge_ids_hbm.at[pl.ds((bstart + csum[nxt]) * _BLK,
                                          _CHUNK_SIZES[nxt] * _BLK)],
                    eidx[b].at[pl.ds(0, _CHUNK_SIZES[nxt] * _BLK)], seid[b])
            cb0 = bstart + csum[c]
            wr[b] = tuple(
                pltpu.async_copy(
                    cbuf[b].at[pl.ds(tj * _CBUF_PLANE, tb * _BLK * 8)],
                    edge_out.at[pl.ds(tj * _PLANE_STRIDE + cb0 * (_BLK * 8),
                                      tb * _BLK * 8)],
                    sew[b])
                for tj in range(2)
            )
            if c == 1:
                ng.wait()
                nw = pltpu.async_copy(
                    nrows_v, node_out.at[pl.ds(nbase, _NODE_CHUNK)], sn_w)
        for hs in wr:
            for h in hs:
                h.wait()
        nw.wait()

    return k(node_ids, edge_ids, node_table, edge_table)


def kernel(node_ids, edge_ids, node_table, edge_table):
    etab_rep = jnp.repeat(edge_table.reshape(-1), _L)
    node_feat, edge_flat = _sc_lookup(node_ids, edge_ids, node_table,
                                      etab_rep)
    edge_feat = (edge_flat.reshape(2, _N_BLKS, 8, _BLK)
                 .transpose(1, 3, 0, 2)
                 .reshape(_N_EDGES, _E_DIM))
    return node_feat, edge_feat


# speedup vs baseline: 1.2694x; 1.0594x over previous
import functools

import jax
import jax.numpy as jnp
from jax import lax
from jax.experimental import pallas as pl
from jax.experimental.pallas import tpu as pltpu
from jax.experimental.pallas import tpu_sc as plsc

_N_NODES = 10000
_N_EDGES = 320000
_NODE_DIM = 128
_E_DIM = 16
_E_VOCAB = 16

_INFO = plsc.get_sparse_core_info()
_NC, _NS = _INFO.num_cores, _INFO.num_subcores
_NW = _NC * _NS
_L = 16

_NODE_CHUNK = 320

_BLK = 128
_N_BLKS = _N_EDGES // _BLK
_NB_W = 79
_TB = 16
_CHUNK_SIZES = (16, 16, 16, 16, 15)
_CBUF_PLANE = _TB * _BLK * 8
_PLANE_STRIDE = _N_BLKS * _BLK * 8
_UNROLL = 4


def _sc_lookup(node_ids, edge_ids, node_table, edge_table):
    mesh = plsc.VectorSubcoreMesh(core_axis_name="c", subcore_axis_name="s")

    @functools.partial(
        pl.kernel,
        mesh=mesh,
        out_type=(
            jax.ShapeDtypeStruct((_N_NODES, _NODE_DIM), jnp.float32),
            jax.ShapeDtypeStruct((_N_EDGES * _E_DIM,), jnp.float32),
        ),
        scratch_types=[
            pltpu.VMEM((_NODE_CHUNK,), jnp.int32),
            pltpu.VMEM((_NODE_CHUNK, _NODE_DIM), jnp.float32),
            pltpu.VMEM((_TB * _BLK,), jnp.int32),
            pltpu.VMEM((_TB * _BLK,), jnp.int32),
            pltpu.VMEM((2 * _CBUF_PLANE,), jnp.float32),
            pltpu.VMEM((2 * _CBUF_PLANE,), jnp.float32),
            pltpu.VMEM((_E_VOCAB * _E_DIM * _L,), jnp.float32),
            pltpu.SemaphoreType.DMA,
            pltpu.SemaphoreType.DMA,
            pltpu.SemaphoreType.DMA,
            pltpu.SemaphoreType.DMA,
            pltpu.SemaphoreType.DMA,
            pltpu.SemaphoreType.DMA,
            pltpu.SemaphoreType.DMA,
        ],
        compiler_params=pltpu.CompilerParams(needs_layout_passes=False),
    )
    def k(node_ids_hbm, edge_ids_hbm, node_tab_hbm, edge_tab_hbm,
          node_out, edge_out, nidx_v, nrows_v, eidx0, eidx1, cbuf0, cbuf1,
          etab_v, sn_id, sn_g, sn_w, se_id0, se_id1, se_w0, se_w1):
        wid = lax.axis_index("s") * _NC + lax.axis_index("c")
        nbase = jnp.where(wid < _NW - 1,
                          wid * _NODE_CHUNK, _N_NODES - _NODE_CHUNK)
        bstart = (wid * _N_BLKS) // _NW

        nid_cp = pltpu.async_copy(
            node_ids_hbm.at[pl.ds(nbase, _NODE_CHUNK)], nidx_v, sn_id)
        eidx = (eidx0, eidx1)
        cbuf = (cbuf0, cbuf1)
        seid = (se_id0, se_id1)
        sew = (se_w0, se_w1)
        csum = [sum(_CHUNK_SIZES[:n]) for n in range(len(_CHUNK_SIZES))]
        ecp = [
            pltpu.async_copy(
                edge_ids_hbm.at[pl.ds((bstart + csum[c]) * _BLK,
                                      _CHUNK_SIZES[c] * _BLK)],
                eidx[c].at[pl.ds(0, _CHUNK_SIZES[c] * _BLK)], seid[c])
            for c in range(2)
        ]
        pltpu.sync_copy(edge_tab_hbm, etab_v)
        nid_cp.wait()
        ng = pltpu.async_copy(node_tab_hbm.at[nidx_v], nrows_v, sn_g)

        lanes = lax.iota(jnp.int32, _L)

        def expand_chunk(ids_ref, rows_ref, n_groups):
            def gather_group(g):
                idvec = ids_ref[pl.ds(g * _L, _L)]
                src = idvec * (_E_DIM * _L) + lanes
                return tuple(plsc.load_gather(etab_v, [src + j * _L])
                             for j in range(_E_DIM))

            def store_group(g, vals):
                dstb = (g // 8) * (8 * _BLK) + (g % 8) * _L
                for j in range(_E_DIM):
                    off = dstb + (j // 8) * _CBUF_PLANE + (j % 8) * _BLK
                    rows_ref[pl.ds(off, _L)] = vals[j]

            def body(it, _):
                g0 = it * _UNROLL
                prev = gather_group(g0)
                for u in range(1, _UNROLL):
                    cur = gather_group(g0 + u)
                    store_group(g0 + u - 1, prev)
                    prev = cur
                store_group(g0 + _UNROLL - 1, prev)
                return 0

            lax.fori_loop(0, n_groups // _UNROLL, body, 0)

        wr = [(), ()]
        nw = None
        n_chunks = len(_CHUNK_SIZES)
        for c in range(n_chunks):
            b = c & 1
            tb = _CHUNK_SIZES[c]
            ecp[b].wait()
            for h in wr[b]:
                h.wait()
            expand_chunk(eidx[b], cbuf[b], tb * 8)
            if c + 2 < n_chunks:
                nxt = c + 2
                ecp[b] = pltpu.async_copy(
                    edge_ids_hbm.at[pl.ds((bstart + csum[nxt]) * _BLK,
                                          _CHUNK_SIZES[nxt] * _BLK)],
                    eidx[b].at[pl.ds(0, _CHUNK_SIZES[nxt] * _BLK)], seid[b])
            cb0 = bstart + csum[c]
            wr[b] = tuple(
                pltpu.async_copy(
                    cbuf[b].at[pl.ds(tj * _CBUF_PLANE, tb * _BLK * 8)],
                    edge_out.at[pl.ds(tj * _PLANE_STRIDE + cb0 * (_BLK * 8),
                                      tb * _BLK * 8)],
                    sew[b])
                for tj in range(2)
            )
            if c == 1:
                ng.wait()
                nw = pltpu.async_copy(
                    nrows_v, node_out.at[pl.ds(nbase, _NODE_CHUNK)], sn_w)
        for hs in wr:
            for h in hs:
                h.wait()
        nw.wait()

    return k(node_ids, edge_ids, node_table, edge_table)


def kernel(node_ids, edge_ids, node_table, edge_table):
    etab_rep = jnp.repeat(edge_table.reshape(-1), _L)
    node_feat, edge_flat = _sc_lookup(node_ids, edge_ids, node_table,
                                      etab_rep)
    edge_feat = (edge_flat.reshape(2, _N_BLKS, 8, _BLK)
                 .transpose(1, 3, 0, 2)
                 .reshape(_N_EDGES, _E_DIM))
    return node_feat, edge_feat
